# baseline (speedup 1.0000x reference)
import jax
import jax.numpy as jnp
from jax.experimental import pallas as pl

B, K, L = 4096, 64, 200
KB = 2


BB2 = 2048


def _body(x_ref, o_ref):
    xi = x_ref[...].astype(jnp.int32)
    kbase = pl.program_id(0) * KB
    k = kbase + jax.lax.broadcasted_iota(jnp.int32, (KB, L, BB2), 0)
    o_ref[...] = jnp.where(k == xi, 0.0, -jnp.inf)


def kernel(x):
    xt = jnp.transpose(x, (1, 2, 0))
    out = pl.pallas_call(
        _body,
        grid=(K // KB, B // BB2),
        in_specs=[pl.BlockSpec((1, L, BB2), lambda i, j: (0, 0, j))],
        out_specs=pl.BlockSpec((KB, L, BB2), lambda i, j: (i, 0, j)),
        out_shape=jax.ShapeDtypeStruct((K, L, B), jnp.float32),
    )(xt)
    return jnp.transpose(out, (2, 0, 1))


# speedup vs baseline: 1.6749x; 1.6749x over previous
import jax
import jax.numpy as jnp
from jax.experimental import pallas as pl

B, K, L = 4096, 64, 200
KB = 2


def _body(x_ref, o_ref):
    xi = x_ref[...].astype(jnp.int32)
    kbase = pl.program_id(0) * KB
    k = kbase + jax.lax.broadcasted_iota(jnp.int32, (KB, L, B), 0)
    o_ref[...] = jnp.where(k == xi, 0.0, -jnp.inf)


def kernel(x):
    xt = jnp.transpose(x, (1, 2, 0))
    out = pl.pallas_call(
        _body,
        grid=(K // KB,),
        in_specs=[pl.BlockSpec((1, L, B), lambda i: (0, 0, 0))],
        out_specs=pl.BlockSpec((KB, L, B), lambda i: (i, 0, 0)),
        out_shape=jax.ShapeDtypeStruct((K, L, B), jnp.float32),
    )(xt)
    return jnp.transpose(out, (2, 0, 1))
